# Initial kernel scaffold; baseline (speedup 1.0000x reference)
import jax
import jax.numpy as jnp
from jax.experimental import pallas as pl


def kernel(phi_a, phi_a_offsets, emb_table, h_bias, lin_w, lin_b):
    raise NotImplementedError("write your pallas kernel here")


# speedup vs baseline: 116.9627x; 116.9627x over previous
import functools

import jax
import jax.numpy as jnp
from jax import lax
from jax.experimental import pallas as pl
from jax.experimental.pallas import tpu as pltpu
from jax.experimental.pallas import tpu_sc as plsc

_CHUNK = 512


def _make_bag_kernel(vocab, d, total, nment, nc, ns):
    nw = nc * ns
    per_w = total // nw
    nchunk = per_w // _CHUNK
    rows_per_sub = nment // ns
    mesh = plsc.VectorSubcoreMesh(core_axis_name="c", subcore_axis_name="s")

    @functools.partial(
        pl.kernel,
        mesh=mesh,
        compiler_params=pltpu.CompilerParams(use_tc_tiling_on_sc=False),
        out_type=jax.ShapeDtypeStruct((nc, nment, d), jnp.float32),
        scratch_types=[
            pltpu.VMEM((_CHUNK,), jnp.int32),
            pltpu.VMEM((_CHUNK,), jnp.int32),
            pltpu.VMEM((_CHUNK, d), jnp.float32),
            pltpu.VMEM_SHARED((nment + 1, d), jnp.float32),
            pltpu.SemaphoreType.DMA,
        ],
    )
    def bag_kernel(table_hbm, idx_hbm, seg_hbm, zeros_hbm, out_hbm,
                   idx_v, seg_v, rows_v, acc, sem):
        cid = lax.axis_index("c")
        sid = lax.axis_index("s")
        wid = sid * nc + cid
        base = wid * per_w

        pltpu.sync_copy(zeros_hbm, acc.at[pl.ds(sid * rows_per_sub,
                                                rows_per_sub)])
        plsc.subcore_barrier()

        def body(i, carry):
            off = pl.multiple_of(base + i * _CHUNK, 8)
            pltpu.sync_copy(idx_hbm.at[pl.ds(off, _CHUNK)], idx_v)
            pltpu.sync_copy(seg_hbm.at[pl.ds(off, _CHUNK)], seg_v)
            pltpu.async_copy(table_hbm.at[idx_v], rows_v, sem).wait()
            pltpu.sync_copy(rows_v, acc.at[seg_v], add=True)
            return carry

        lax.fori_loop(0, nchunk, body, 0)
        plsc.subcore_barrier()

        sl = pl.ds(sid * rows_per_sub, rows_per_sub)
        pltpu.sync_copy(acc.at[sl], out_hbm.at[cid, sl])

    return bag_kernel


def _head_body(p_ref, bias_ref, w_ref, b_ref, ha_ref, sc_ref):
    bags = p_ref[0] + p_ref[1]
    h = jnp.tanh(bags + bias_ref[...])
    ha_ref[...] = h
    sc_ref[...] = jnp.sum(h * w_ref[...], axis=1, keepdims=True) + b_ref[...]


def kernel(phi_a, phi_a_offsets, emb_table, h_bias, lin_w, lin_b):
    total = phi_a.shape[0]
    nment = phi_a_offsets.shape[0] - 1
    vocab, d = emb_table.shape

    offs = phi_a_offsets.astype(jnp.int32)
    ids = jnp.arange(nment + 1, dtype=jnp.int32)
    marks = jnp.full((total,), -1, jnp.int32).at[offs].max(
        ids, mode="drop", indices_are_sorted=True)
    rm = marks.reshape(total // 128, 128)
    row_pre = lax.cummax(rm, axis=1)
    carry = lax.cummax(jnp.max(rm, axis=1), axis=0)
    carry_excl = jnp.concatenate(
        [jnp.full((1,), -1, jnp.int32), carry[:-1]])
    seg = jnp.maximum(row_pre, carry_excl[:, None]).reshape(total)
    seg = jnp.where(seg < 0, nment, seg)
    idx = phi_a.astype(jnp.int32)

    info = plsc.get_sparse_core_info()
    nc, ns = info.num_cores, info.num_subcores
    zeros = jnp.zeros((nment // ns, d), jnp.float32)

    bag_kernel = _make_bag_kernel(vocab, d, total, nment, nc, ns)
    partials = bag_kernel(emb_table, idx, seg, zeros)

    ha, scores = pl.pallas_call(
        _head_body,
        out_shape=(
            jax.ShapeDtypeStruct((nment, d), jnp.float32),
            jax.ShapeDtypeStruct((nment, 1), jnp.float32),
        ),
    )(partials, h_bias.reshape(1, d), lin_w, lin_b.reshape(1, 1))
    return scores[:, 0], ha
